# baseline (speedup 1.0000x reference)
import functools

import jax
import jax.numpy as jnp
from jax import lax
from jax.experimental import pallas as pl
from jax.experimental.pallas import tpu as pltpu
from jax.experimental.pallas import tpu_sc as plsc

_SEQ = 100000
_J = 75
_D = 3 * _J
_LANES = 16
_WIN = 256
_BMAX = (_SEQ // 128) * 128 - 128
_NTILES = 15

_mesh = plsc.VectorSubcoreMesh(
    core_axis_name="c", subcore_axis_name="s", num_cores=1, num_subcores=16
)


@functools.partial(
    pl.kernel,
    out_type=jax.ShapeDtypeStruct((_NTILES * _LANES,), jnp.float32),
    mesh=_mesh,
    scratch_types=[
        pltpu.VMEM((_LANES,), jnp.float32),
        pltpu.VMEM((1, _LANES, _WIN), jnp.float32),
        pltpu.VMEM((_LANES,), jnp.float32),
        pltpu.VMEM((2 * _LANES,), jnp.float32),
        pltpu.SemaphoreType.DMA,
    ],
)
def _spline_sc(time_hbm, table_hbm, out_hbm, time_v, win_v, res_v, red_v, sem):
    wid = lax.axis_index("s")

    @pl.when(wid < _NTILES)
    def _():
        lane = lax.iota(jnp.int32, _LANES)
        d = wid // 5
        j0 = pl.multiple_of((wid % 5) * _LANES, _LANES)

        pltpu.sync_copy(time_hbm, time_v.at[pl.ds(0, 1)])
        t = time_v[...][0]
        scaled = jnp.clip(t, 0.0, 1.0) * jnp.float32(_SEQ - 1)
        i_rn = scaled.astype(jnp.int32)
        i = i_rn - (i_rn.astype(jnp.float32) > scaled).astype(jnp.int32)
        s = scaled - i.astype(jnp.float32)

        b0 = jnp.clip(i - 1, 0, _SEQ - 1)
        base = jnp.minimum((b0 // 128) * 128, _BMAX)
        base = pl.multiple_of(base, 128)
        pltpu.async_copy(
            table_hbm.at[pl.ds(d, 1), pl.ds(j0, _LANES), pl.ds(base, _WIN)],
            win_v,
            sem,
        ).wait()

        s2 = s * s
        s3 = s2 * s
        w = [
            0.5 * (-s + 2.0 * s2 - s3),
            0.5 * (2.0 - 5.0 * s2 + 3.0 * s3),
            0.5 * (s + 4.0 * s2 - 3.0 * s3),
            0.5 * (-s2 + s3),
        ]
        col = [jnp.clip(i + (k - 1), 0, _SEQ - 1) - base for k in range(4)]

        c0 = col[0]
        off = pl.multiple_of((c0 // _LANES) * _LANES, _LANES)
        zeros = jnp.zeros((_LANES,), jnp.float32)
        clo = off + lane
        chi = clo + _LANES
        wlo = zeros
        whi = zeros
        for k in range(4):
            wlo = wlo + jnp.where(clo == col[k], w[k], 0.0)
            whi = whi + jnp.where(chi == col[k], w[k], 0.0)
        hi_used = whi != 0.0

        res = zeros
        for jj in range(_LANES):
            vlo = win_v[0, jj, pl.ds(off, _LANES)]
            vhi = win_v[0, jj, pl.ds(off + _LANES, _LANES)]
            acc = vlo * wlo + jnp.where(hi_used, vhi * whi, 0.0)
            for shift in (8, 4, 2, 1):
                red_v[pl.ds(0, _LANES)] = acc
                acc = acc + red_v[pl.ds(shift, _LANES)]
            res = jnp.where(lane == jj, acc[0], res)
        res_v[...] = res

        slot = pl.multiple_of(wid * _LANES, _LANES)
        pltpu.sync_copy(res_v, out_hbm.at[pl.ds(slot, _LANES)])


def kernel(time_point, control_points):
    table = control_points.transpose(2, 1, 0)
    flat = _spline_sc(time_point, table)
    return flat.reshape(3, 5 * _LANES)[:, :_J].transpose(1, 0).reshape(1, _J, 3)


# speedup vs baseline: 1.0171x; 1.0171x over previous
import functools

import jax
import jax.numpy as jnp
from jax import lax
from jax.experimental import pallas as pl
from jax.experimental.pallas import tpu as pltpu
from jax.experimental.pallas import tpu_sc as plsc

_SEQ = 100000
_J = 75
_D = 3 * _J
_LANES = 16
_WIN = 256
_BMAX = (_SEQ // 128) * 128 - 128
_NTILES = 15

_mesh = plsc.VectorSubcoreMesh(
    core_axis_name="c", subcore_axis_name="s", num_cores=1, num_subcores=16
)


@functools.partial(
    pl.kernel,
    out_type=jax.ShapeDtypeStruct((_NTILES * _LANES,), jnp.float32),
    mesh=_mesh,
    scratch_types=[
        pltpu.VMEM((_LANES,), jnp.float32),
        pltpu.VMEM((1, _LANES, _WIN), jnp.float32),
        pltpu.VMEM((_LANES,), jnp.float32),
        pltpu.VMEM((_LANES * 2 * _LANES,), jnp.float32),
        pltpu.SemaphoreType.DMA,
    ],
)
def _spline_sc(time_hbm, table_hbm, out_hbm, time_v, win_v, res_v, red_v, sem):
    wid = lax.axis_index("s")

    @pl.when(wid < _NTILES)
    def _():
        lane = lax.iota(jnp.int32, _LANES)
        d = wid // 5
        j0 = pl.multiple_of((wid % 5) * _LANES, _LANES)

        pltpu.sync_copy(time_hbm, time_v.at[pl.ds(0, 1)])
        t = time_v[...][0]
        scaled = jnp.clip(t, 0.0, 1.0) * jnp.float32(_SEQ - 1)
        i_rn = scaled.astype(jnp.int32)
        i = i_rn - (i_rn.astype(jnp.float32) > scaled).astype(jnp.int32)
        s = scaled - i.astype(jnp.float32)

        b0 = jnp.clip(i - 1, 0, _SEQ - 1)
        base = jnp.minimum((b0 // 128) * 128, _BMAX)
        base = pl.multiple_of(base, 128)
        win_copy = pltpu.async_copy(
            table_hbm.at[pl.ds(d, 1), pl.ds(j0, _LANES), pl.ds(base, _WIN)],
            win_v,
            sem,
        )

        s2 = s * s
        s3 = s2 * s
        w = [
            0.5 * (-s + 2.0 * s2 - s3),
            0.5 * (2.0 - 5.0 * s2 + 3.0 * s3),
            0.5 * (s + 4.0 * s2 - 3.0 * s3),
            0.5 * (-s2 + s3),
        ]
        col = [jnp.clip(i + (k - 1), 0, _SEQ - 1) - base for k in range(4)]

        c0 = col[0]
        off = pl.multiple_of((c0 // _LANES) * _LANES, _LANES)
        zeros = jnp.zeros((_LANES,), jnp.float32)
        clo = off + lane
        chi = clo + _LANES
        wlo = zeros
        whi = zeros
        for k in range(4):
            wlo = wlo + jnp.where(clo == col[k], w[k], 0.0)
            whi = whi + jnp.where(chi == col[k], w[k], 0.0)
        hi_used = whi != 0.0

        win_copy.wait()
        res = zeros
        for jj in range(_LANES):
            vlo = win_v[0, jj, pl.ds(off, _LANES)]
            vhi = win_v[0, jj, pl.ds(off + _LANES, _LANES)]
            acc = vlo * wlo + jnp.where(hi_used, vhi * whi, 0.0)
            rbase = jj * 2 * _LANES
            for shift in (8, 4, 2, 1):
                red_v[pl.ds(rbase, _LANES)] = acc
                acc = acc + red_v[pl.ds(rbase + shift, _LANES)]
            res = jnp.where(lane == jj, acc[0], res)
        res_v[...] = res

        slot = pl.multiple_of(wid * _LANES, _LANES)
        pltpu.sync_copy(res_v, out_hbm.at[pl.ds(slot, _LANES)])


def kernel(time_point, control_points):
    table = control_points.transpose(2, 1, 0)
    flat = _spline_sc(time_point, table)
    return flat.reshape(3, 5 * _LANES)[:, :_J].transpose(1, 0).reshape(1, _J, 3)
